# baseline (speedup 1.0000x reference)
import functools

import jax
import jax.numpy as jnp
from jax.experimental import pallas as pl
from jax.experimental.pallas import tpu as pltpu

NUM_HEADS = 16
NUM_EXPERTS = 8
TOP_K = 2
HIDDEN = 1024
FFN = 4096
EPS = 1e-5

S = 2048
DH = HIDDEN // NUM_HEADS
BQ = 512
BK = 512
BR = 256
BLK = 256
NSLOT = S * TOP_K
NROWS = ((NSLOT + NUM_EXPERTS * (BLK - 1) + BLK - 1) // BLK) * BLK
NBLK = NROWS // BLK


def _ln(x, w, b):
    mu = jnp.mean(x, axis=-1, keepdims=True)
    var = jnp.mean((x - mu) ** 2, axis=-1, keepdims=True)
    return (x - mu) / jnp.sqrt(var + EPS) * w + b


def _ln1_qkv_body(x_ref, w_ref, lnw_ref, lnb_ref, q_ref, k_ref, v_ref):
    x = x_ref[...]
    ln = _ln(x, lnw_ref[...], lnb_ref[...]).astype(jnp.bfloat16)
    qkv = jax.lax.dot_general(
        ln, w_ref[...], (((1,), (1,)), ((), ())),
        preferred_element_type=jnp.float32).astype(jnp.bfloat16)
    for ref, off in ((q_ref, 0), (k_ref, HIDDEN), (v_ref, 2 * HIDDEN)):
        part = qkv[:, off:off + HIDDEN]
        ref[...] = part.reshape(BR, NUM_HEADS, DH).transpose(1, 0, 2)


def _attn_body(q_ref, k_ref, v_ref, o_ref):
    qi = pl.program_id(1)
    q = q_ref[0] * jnp.bfloat16(1.0 / DH ** 0.5)
    row = jax.lax.broadcasted_iota(jnp.int32, (BQ, BK), 0)
    col = jax.lax.broadcasted_iota(jnp.int32, (BQ, BK), 1)
    diag_mask = col <= row

    def tile(t, carry):
        acc, l = carry
        kt = k_ref[0, pl.ds(t * BK, BK), :]
        s = jax.lax.dot_general(q, kt, (((1,), (1,)), ((), ())),
                                preferred_element_type=jnp.float32)
        e = jnp.exp(s)
        p = jax.lax.cond(t == qi,
                         lambda e: jnp.where(diag_mask, e, 0.0),
                         lambda e: e, e)
        l = l + jnp.sum(p, axis=1, keepdims=True)
        vt = v_ref[0, pl.ds(t * BK, BK), :]
        acc = acc + jax.lax.dot_general(p.astype(jnp.bfloat16), vt,
                                        (((1,), (0,)), ((), ())),
                                        preferred_element_type=jnp.float32)
        return acc, l

    acc0 = jnp.zeros((BQ, DH), jnp.float32)
    l0 = jnp.zeros((BQ, 1), jnp.float32)
    acc, l = jax.lax.fori_loop(0, qi + 1, tile, (acc0, l0))
    o_ref[0] = (acc / l).astype(jnp.bfloat16)


def _proj_ln2_body(ao_ref, h_ref, pw_ref, lnw_ref, lnb_ref, rw_ref,
                   ha_ref, x2_ref, lg_ref):
    ao = ao_ref[...].transpose(1, 0, 2).reshape(BR, HIDDEN)
    proj = jax.lax.dot_general(ao, pw_ref[...],
                               (((1,), (1,)), ((), ())),
                               preferred_element_type=jnp.float32)
    ha = h_ref[...] + proj
    ha_ref[...] = ha
    x2 = _ln(ha, lnw_ref[...], lnb_ref[...])
    x2_ref[...] = x2.astype(jnp.bfloat16)
    lg_ref[...] = jax.lax.dot_general(x2.astype(jnp.bfloat16), rw_ref[...],
                                      (((1,), (1,)), ((), ())),
                                      preferred_element_type=jnp.float32)


def _ffn_body(meta_ref, xs_ref, w1_ref, w2_ref, g_ref, ys_ref):
    i = pl.program_id(1)

    @pl.when(i < meta_ref[NBLK])
    def _():
        xs = xs_ref[...].astype(jnp.float32)
        h = jax.lax.dot_general(xs, w1_ref[0], (((1,), (1,)), ((), ())),
                                preferred_element_type=jnp.float32)
        h = jax.nn.gelu(h)
        o = jax.lax.dot_general(h, w2_ref[0], (((1,), (1,)), ((), ())),
                                preferred_element_type=jnp.float32)
        g = g_ref[0, 0, :]
        ys_ref[0] = (o * g[:, None]).astype(jnp.bfloat16)


def _combine_body(ha_ref, a_ref, b_ref, c_ref, d_ref, out_ref):
    f = jnp.float32
    out_ref[...] = (ha_ref[...]
                    + a_ref[...].astype(f) + b_ref[...].astype(f)
                    + c_ref[...].astype(f) + d_ref[...].astype(f))


def kernel(hidden_states, ln1_weight, ln1_bias, ln2_weight, ln2_bias,
           qkv_weight, proj_weight, router_weight, moe_w1, moe_w2):
    x = hidden_states.reshape(S, HIDDEN)
    f32 = jnp.float32

    ln1w = ln1_weight.reshape(1, HIDDEN)
    ln1b = ln1_bias.reshape(1, HIDDEN)
    ln2w = ln2_weight.reshape(1, HIDDEN)
    ln2b = ln2_bias.reshape(1, HIDDEN)

    hd = jax.ShapeDtypeStruct((NUM_HEADS, S, DH), jnp.bfloat16)
    q, k, v = pl.pallas_call(
        _ln1_qkv_body,
        grid=(S // BR,),
        in_specs=[
            pl.BlockSpec((BR, HIDDEN), lambda i: (i, 0)),
            pl.BlockSpec((3 * HIDDEN, HIDDEN), lambda i: (0, 0)),
            pl.BlockSpec((1, HIDDEN), lambda i: (0, 0)),
            pl.BlockSpec((1, HIDDEN), lambda i: (0, 0)),
        ],
        out_specs=[pl.BlockSpec((NUM_HEADS, BR, DH), lambda i: (0, i, 0))] * 3,
        out_shape=[hd, hd, hd],
    )(x, qkv_weight.astype(jnp.bfloat16), ln1w, ln1b)

    ao = pl.pallas_call(
        _attn_body,
        grid=(NUM_HEADS, S // BQ),
        in_specs=[
            pl.BlockSpec((1, BQ, DH), lambda h, i: (h, i, 0)),
            pl.BlockSpec((1, S, DH), lambda h, i: (h, 0, 0)),
            pl.BlockSpec((1, S, DH), lambda h, i: (h, 0, 0)),
        ],
        out_specs=pl.BlockSpec((1, BQ, DH), lambda h, i: (h, i, 0)),
        out_shape=jax.ShapeDtypeStruct((NUM_HEADS, S, DH), jnp.bfloat16),
    )(q, k, v)

    ha, x2b, logits = pl.pallas_call(
        _proj_ln2_body,
        grid=(S // BR,),
        in_specs=[
            pl.BlockSpec((NUM_HEADS, BR, DH), lambda i: (0, i, 0)),
            pl.BlockSpec((BR, HIDDEN), lambda i: (i, 0)),
            pl.BlockSpec((HIDDEN, HIDDEN), lambda i: (0, 0)),
            pl.BlockSpec((1, HIDDEN), lambda i: (0, 0)),
            pl.BlockSpec((1, HIDDEN), lambda i: (0, 0)),
            pl.BlockSpec((NUM_EXPERTS, HIDDEN), lambda i: (0, 0)),
        ],
        out_specs=[
            pl.BlockSpec((BR, HIDDEN), lambda i: (i, 0)),
            pl.BlockSpec((BR, HIDDEN), lambda i: (i, 0)),
            pl.BlockSpec((BR, NUM_EXPERTS), lambda i: (i, 0)),
        ],
        out_shape=[
            jax.ShapeDtypeStruct((S, HIDDEN), f32),
            jax.ShapeDtypeStruct((S, HIDDEN), jnp.bfloat16),
            jax.ShapeDtypeStruct((S, NUM_EXPERTS), f32),
        ],
    )(ao, x, proj_weight.astype(jnp.bfloat16), ln2w, ln2b,
      router_weight.astype(jnp.bfloat16))

    probs = jax.nn.softmax(logits, axis=-1)
    top_p, top_i = jax.lax.top_k(probs, TOP_K)
    flat_e = top_i.reshape(-1).astype(jnp.int32)
    onehot = (flat_e[:, None] == jnp.arange(NUM_EXPERTS)[None, :]
              ).astype(jnp.int32)
    counts = onehot.sum(0)
    nblk_e = (counts + BLK - 1) // BLK
    padded = nblk_e * BLK
    ends = jnp.cumsum(padded)
    offs = ends - padded
    rank = jnp.cumsum(onehot, axis=0) - onehot
    myrank = (rank * onehot).sum(1)
    pos = offs[flat_e] + myrank
    slot_tok = jnp.arange(NSLOT, dtype=jnp.int32) // TOP_K
    tids = jnp.zeros((NROWS,), jnp.int32).at[pos].set(slot_tok)
    gates = jnp.zeros((NROWS,), f32).at[pos].set(top_p.reshape(-1))
    blk_start = jnp.arange(NBLK, dtype=jnp.int32) * BLK
    block_expert = (blk_start[:, None] >= ends[None, :]).sum(1)
    block_expert = jnp.minimum(block_expert, NUM_EXPERTS - 1).astype(jnp.int32)
    num_used = nblk_e.sum().astype(jnp.int32)
    meta = jnp.concatenate([block_expert, num_used[None]])

    xs = x2b[tids]
    gates3 = gates.reshape(NBLK, 1, BLK)
    FC = FFN // 2
    NFC = FFN // FC

    ysp = pl.pallas_call(
        _ffn_body,
        grid_spec=pltpu.PrefetchScalarGridSpec(
            num_scalar_prefetch=1,
            grid=(NFC, NBLK),
            in_specs=[
                pl.BlockSpec((BLK, HIDDEN), lambda fc, i, m: (i, 0)),
                pl.BlockSpec((1, FC, HIDDEN), lambda fc, i, m: (m[i], fc, 0)),
                pl.BlockSpec((1, HIDDEN, FC), lambda fc, i, m: (m[i], 0, fc)),
                pl.BlockSpec((1, 1, BLK), lambda fc, i, m: (i, 0, 0)),
            ],
            out_specs=pl.BlockSpec((1, BLK, HIDDEN), lambda fc, i, m: (fc, i, 0)),
        ),
        out_shape=jax.ShapeDtypeStruct((NFC, NROWS, HIDDEN), jnp.bfloat16),
    )(meta, xs, moe_w1, moe_w2, gates3)

    flat = ysp.reshape(NFC * NROWS, HIDDEN)
    pos2 = pos.reshape(S, TOP_K)
    allpos = jnp.concatenate(
        [pos2[:, 0], pos2[:, 1], pos2[:, 0] + NROWS, pos2[:, 1] + NROWS])
    zz = flat[allpos]
    out = pl.pallas_call(
        _combine_body,
        grid=(S // BR,),
        in_specs=[
            pl.BlockSpec((BR, HIDDEN), lambda i: (i, 0)),
            pl.BlockSpec((BR, HIDDEN), lambda i: (i, 0)),
            pl.BlockSpec((BR, HIDDEN), lambda i: (S // BR + i, 0)),
            pl.BlockSpec((BR, HIDDEN), lambda i: (2 * (S // BR) + i, 0)),
            pl.BlockSpec((BR, HIDDEN), lambda i: (3 * (S // BR) + i, 0)),
        ],
        out_specs=pl.BlockSpec((BR, HIDDEN), lambda i: (i, 0)),
        out_shape=jax.ShapeDtypeStruct((S, HIDDEN), f32),
    )(ha, zz, zz, zz, zz)

    return out.reshape(S, 1, HIDDEN)


# speedup vs baseline: 1.0772x; 1.0772x over previous
import functools

import jax
import jax.numpy as jnp
from jax.experimental import pallas as pl
from jax.experimental.pallas import tpu as pltpu

NUM_HEADS = 16
NUM_EXPERTS = 8
TOP_K = 2
HIDDEN = 1024
FFN = 4096
EPS = 1e-5

S = 2048
DH = HIDDEN // NUM_HEADS
BQ = 512
BK = 512
BR = 256
BLK = 256
NSLOT = S * TOP_K
NROWS = ((NSLOT + NUM_EXPERTS * (BLK - 1) + BLK - 1) // BLK) * BLK
NBLK = NROWS // BLK


def _ln(x, w, b):
    mu = jnp.mean(x, axis=-1, keepdims=True)
    var = jnp.mean((x - mu) ** 2, axis=-1, keepdims=True)
    return (x - mu) / jnp.sqrt(var + EPS) * w + b


def _ln1_qkv_body(x_ref, w_ref, lnw_ref, lnb_ref, q_ref, k_ref, v_ref):
    x = x_ref[...]
    ln = _ln(x, lnw_ref[...], lnb_ref[...]).astype(jnp.bfloat16)
    qkv = jax.lax.dot_general(
        ln, w_ref[...], (((1,), (1,)), ((), ())),
        preferred_element_type=jnp.float32).astype(jnp.bfloat16)
    for ref, off in ((q_ref, 0), (k_ref, HIDDEN), (v_ref, 2 * HIDDEN)):
        part = qkv[:, off:off + HIDDEN]
        ref[...] = part.reshape(BR, NUM_HEADS, DH).transpose(1, 0, 2)


def _attn_body(q_ref, k_ref, v_ref, o_ref):
    qi = pl.program_id(1)
    q = q_ref[0] * jnp.bfloat16(1.0 / DH ** 0.5)
    row = jax.lax.broadcasted_iota(jnp.int32, (BQ, BK), 0)
    col = jax.lax.broadcasted_iota(jnp.int32, (BQ, BK), 1)
    cmr = col - row

    def tile(t, carry):
        acc, l = carry
        kt = k_ref[0, pl.ds(t * BK, BK), :]
        s = jax.lax.dot_general(q, kt, (((1,), (1,)), ((), ())),
                                preferred_element_type=jnp.float32)
        p = jnp.where(cmr <= (qi - t) * BK, jnp.exp(s), 0.0)
        l = l + jnp.sum(p, axis=1, keepdims=True)
        vt = v_ref[0, pl.ds(t * BK, BK), :]
        acc = acc + jax.lax.dot_general(p.astype(jnp.bfloat16), vt,
                                        (((1,), (0,)), ((), ())),
                                        preferred_element_type=jnp.float32)
        return acc, l

    acc0 = jnp.zeros((BQ, DH), jnp.float32)
    l0 = jnp.zeros((BQ, 1), jnp.float32)
    acc, l = jax.lax.fori_loop(0, qi + 1, tile, (acc0, l0))
    o_ref[0] = (acc / l).astype(jnp.bfloat16)


def _proj_ln2_body(ao_ref, h_ref, pw_ref, lnw_ref, lnb_ref, rw_ref,
                   ha_ref, x2_ref, lg_ref):
    ao = ao_ref[...].transpose(1, 0, 2).reshape(BR, HIDDEN)
    proj = jax.lax.dot_general(ao, pw_ref[...],
                               (((1,), (1,)), ((), ())),
                               preferred_element_type=jnp.float32)
    ha = h_ref[...] + proj
    ha_ref[...] = ha
    x2 = _ln(ha, lnw_ref[...], lnb_ref[...])
    x2_ref[...] = x2.astype(jnp.bfloat16)
    lg_ref[...] = jax.lax.dot_general(x2.astype(jnp.bfloat16), rw_ref[...],
                                      (((1,), (1,)), ((), ())),
                                      preferred_element_type=jnp.float32)


def _ffn_body(meta_ref, xs_ref, w1_ref, w2_ref, g_ref, ys_ref):
    i = pl.program_id(1)

    @pl.when(i < meta_ref[NBLK])
    def _():
        xs = xs_ref[...].astype(jnp.float32)
        h = jax.lax.dot_general(xs, w1_ref[0], (((1,), (1,)), ((), ())),
                                preferred_element_type=jnp.float32)
        h = jax.nn.gelu(h)
        o = jax.lax.dot_general(h, w2_ref[0], (((1,), (1,)), ((), ())),
                                preferred_element_type=jnp.float32)
        g = g_ref[0, 0, :]
        ys_ref[0] = (o * g[:, None]).astype(jnp.bfloat16)


def _combine_body(ha_ref, a_ref, b_ref, c_ref, d_ref, out_ref):
    f = jnp.float32
    out_ref[...] = (ha_ref[...]
                    + a_ref[...].astype(f) + b_ref[...].astype(f)
                    + c_ref[...].astype(f) + d_ref[...].astype(f))


def kernel(hidden_states, ln1_weight, ln1_bias, ln2_weight, ln2_bias,
           qkv_weight, proj_weight, router_weight, moe_w1, moe_w2):
    x = hidden_states.reshape(S, HIDDEN)
    f32 = jnp.float32

    ln1w = ln1_weight.reshape(1, HIDDEN)
    ln1b = ln1_bias.reshape(1, HIDDEN)
    ln2w = ln2_weight.reshape(1, HIDDEN)
    ln2b = ln2_bias.reshape(1, HIDDEN)

    hd = jax.ShapeDtypeStruct((NUM_HEADS, S, DH), jnp.bfloat16)
    q, k, v = pl.pallas_call(
        _ln1_qkv_body,
        grid=(S // BR,),
        in_specs=[
            pl.BlockSpec((BR, HIDDEN), lambda i: (i, 0)),
            pl.BlockSpec((3 * HIDDEN, HIDDEN), lambda i: (0, 0)),
            pl.BlockSpec((1, HIDDEN), lambda i: (0, 0)),
            pl.BlockSpec((1, HIDDEN), lambda i: (0, 0)),
        ],
        out_specs=[pl.BlockSpec((NUM_HEADS, BR, DH), lambda i: (0, i, 0))] * 3,
        out_shape=[hd, hd, hd],
    )(x, qkv_weight.astype(jnp.bfloat16), ln1w, ln1b)

    ao = pl.pallas_call(
        _attn_body,
        grid=(NUM_HEADS, S // BQ),
        in_specs=[
            pl.BlockSpec((1, BQ, DH), lambda h, i: (h, i, 0)),
            pl.BlockSpec((1, S, DH), lambda h, i: (h, 0, 0)),
            pl.BlockSpec((1, S, DH), lambda h, i: (h, 0, 0)),
        ],
        out_specs=pl.BlockSpec((1, BQ, DH), lambda h, i: (h, i, 0)),
        out_shape=jax.ShapeDtypeStruct((NUM_HEADS, S, DH), jnp.bfloat16),
    )(q, k, v)

    ha, x2b, logits = pl.pallas_call(
        _proj_ln2_body,
        grid=(S // BR,),
        in_specs=[
            pl.BlockSpec((NUM_HEADS, BR, DH), lambda i: (0, i, 0)),
            pl.BlockSpec((BR, HIDDEN), lambda i: (i, 0)),
            pl.BlockSpec((HIDDEN, HIDDEN), lambda i: (0, 0)),
            pl.BlockSpec((1, HIDDEN), lambda i: (0, 0)),
            pl.BlockSpec((1, HIDDEN), lambda i: (0, 0)),
            pl.BlockSpec((NUM_EXPERTS, HIDDEN), lambda i: (0, 0)),
        ],
        out_specs=[
            pl.BlockSpec((BR, HIDDEN), lambda i: (i, 0)),
            pl.BlockSpec((BR, HIDDEN), lambda i: (i, 0)),
            pl.BlockSpec((BR, NUM_EXPERTS), lambda i: (i, 0)),
        ],
        out_shape=[
            jax.ShapeDtypeStruct((S, HIDDEN), f32),
            jax.ShapeDtypeStruct((S, HIDDEN), jnp.bfloat16),
            jax.ShapeDtypeStruct((S, NUM_EXPERTS), f32),
        ],
    )(ao, x, proj_weight.astype(jnp.bfloat16), ln2w, ln2b,
      router_weight.astype(jnp.bfloat16))

    probs = jax.nn.softmax(logits, axis=-1)
    top_p, top_i = jax.lax.top_k(probs, TOP_K)
    flat_e = top_i.reshape(-1).astype(jnp.int32)
    onehot = (flat_e[:, None] == jnp.arange(NUM_EXPERTS)[None, :]
              ).astype(jnp.int32)
    counts = onehot.sum(0)
    nblk_e = (counts + BLK - 1) // BLK
    padded = nblk_e * BLK
    ends = jnp.cumsum(padded)
    offs = ends - padded
    rank = jnp.cumsum(onehot, axis=0) - onehot
    myrank = (rank * onehot).sum(1)
    pos = offs[flat_e] + myrank
    slot_tok = jnp.arange(NSLOT, dtype=jnp.int32) // TOP_K
    tids = jnp.zeros((NROWS,), jnp.int32).at[pos].set(slot_tok)
    gates = jnp.zeros((NROWS,), f32).at[pos].set(top_p.reshape(-1))
    blk_start = jnp.arange(NBLK, dtype=jnp.int32) * BLK
    block_expert = (blk_start[:, None] >= ends[None, :]).sum(1)
    block_expert = jnp.minimum(block_expert, NUM_EXPERTS - 1).astype(jnp.int32)
    num_used = nblk_e.sum().astype(jnp.int32)
    meta = jnp.concatenate([block_expert, num_used[None]])

    xs = x2b[tids]
    gates3 = gates.reshape(NBLK, 1, BLK)
    FC = FFN // 2
    NFC = FFN // FC

    ysp = pl.pallas_call(
        _ffn_body,
        grid_spec=pltpu.PrefetchScalarGridSpec(
            num_scalar_prefetch=1,
            grid=(NFC, NBLK),
            in_specs=[
                pl.BlockSpec((BLK, HIDDEN), lambda fc, i, m: (i, 0)),
                pl.BlockSpec((1, FC, HIDDEN), lambda fc, i, m: (m[i], fc, 0)),
                pl.BlockSpec((1, HIDDEN, FC), lambda fc, i, m: (m[i], 0, fc)),
                pl.BlockSpec((1, 1, BLK), lambda fc, i, m: (i, 0, 0)),
            ],
            out_specs=pl.BlockSpec((1, BLK, HIDDEN), lambda fc, i, m: (fc, i, 0)),
        ),
        out_shape=jax.ShapeDtypeStruct((NFC, NROWS, HIDDEN), jnp.bfloat16),
    )(meta, xs, moe_w1, moe_w2, gates3)

    flat = ysp.reshape(NFC * NROWS, HIDDEN)
    pos2 = pos.reshape(S, TOP_K)
    allpos = jnp.concatenate(
        [pos2[:, 0], pos2[:, 1], pos2[:, 0] + NROWS, pos2[:, 1] + NROWS])
    zz = flat[allpos]
    out = pl.pallas_call(
        _combine_body,
        grid=(S // BR,),
        in_specs=[
            pl.BlockSpec((BR, HIDDEN), lambda i: (i, 0)),
            pl.BlockSpec((BR, HIDDEN), lambda i: (i, 0)),
            pl.BlockSpec((BR, HIDDEN), lambda i: (S // BR + i, 0)),
            pl.BlockSpec((BR, HIDDEN), lambda i: (2 * (S // BR) + i, 0)),
            pl.BlockSpec((BR, HIDDEN), lambda i: (3 * (S // BR) + i, 0)),
        ],
        out_specs=pl.BlockSpec((BR, HIDDEN), lambda i: (i, 0)),
        out_shape=jax.ShapeDtypeStruct((S, HIDDEN), f32),
    )(ha, zz, zz, zz, zz)

    return out.reshape(S, 1, HIDDEN)


# speedup vs baseline: 1.0990x; 1.0202x over previous
import functools

import jax
import jax.numpy as jnp
from jax.experimental import pallas as pl
from jax.experimental.pallas import tpu as pltpu

NUM_HEADS = 16
NUM_EXPERTS = 8
TOP_K = 2
HIDDEN = 1024
FFN = 4096
EPS = 1e-5

S = 2048
DH = HIDDEN // NUM_HEADS
BQ = 512
BK = 512
BR = 256
BLK = 256
NSLOT = S * TOP_K
NROWS = ((NSLOT + NUM_EXPERTS * (BLK - 1) + BLK - 1) // BLK) * BLK
NBLK = NROWS // BLK


def _ln(x, w, b):
    mu = jnp.mean(x, axis=-1, keepdims=True)
    var = jnp.mean((x - mu) ** 2, axis=-1, keepdims=True)
    return (x - mu) / jnp.sqrt(var + EPS) * w + b


def _ln1_qkv_body(x_ref, w_ref, lnw_ref, lnb_ref, q_ref, k_ref, v_ref):
    x = x_ref[...]
    ln = _ln(x, lnw_ref[...], lnb_ref[...]).astype(jnp.bfloat16)
    qkv = jax.lax.dot_general(
        ln, w_ref[...], (((1,), (1,)), ((), ())),
        preferred_element_type=jnp.float32).astype(jnp.bfloat16)
    for ref, off in ((q_ref, 0), (k_ref, HIDDEN), (v_ref, 2 * HIDDEN)):
        part = qkv[:, off:off + HIDDEN]
        ref[...] = part.reshape(BR, NUM_HEADS, DH).transpose(1, 0, 2)


def _attn_body(q_ref, k_ref, v_ref, o_ref):
    qi = pl.program_id(1)
    q = q_ref[0] * jnp.bfloat16(1.0 / DH ** 0.5)
    row = jax.lax.broadcasted_iota(jnp.int32, (BQ, BK), 0)
    col = jax.lax.broadcasted_iota(jnp.int32, (BQ, BK), 1)
    cmr = col - row

    def tile(t, carry):
        acc, l = carry
        kt = k_ref[0, pl.ds(t * BK, BK), :]
        s = jax.lax.dot_general(q, kt, (((1,), (1,)), ((), ())),
                                preferred_element_type=jnp.float32)
        p = jnp.where(cmr <= (qi - t) * BK, jnp.exp(s), 0.0)
        l = l + jnp.sum(p, axis=1, keepdims=True)
        vt = v_ref[0, pl.ds(t * BK, BK), :]
        acc = acc + jax.lax.dot_general(p.astype(jnp.bfloat16), vt,
                                        (((1,), (0,)), ((), ())),
                                        preferred_element_type=jnp.float32)
        return acc, l

    acc0 = jnp.zeros((BQ, DH), jnp.float32)
    l0 = jnp.zeros((BQ, 1), jnp.float32)
    acc, l = jax.lax.fori_loop(0, qi + 1, tile, (acc0, l0))
    o_ref[0] = (acc / l).astype(jnp.bfloat16)


def _proj_ln2_body(ao_ref, h_ref, pw_ref, lnw_ref, lnb_ref, rw_ref,
                   ha_ref, x2_ref, lg_ref):
    ao = ao_ref[...].transpose(1, 0, 2).reshape(BR, HIDDEN)
    proj = jax.lax.dot_general(ao, pw_ref[...],
                               (((1,), (1,)), ((), ())),
                               preferred_element_type=jnp.float32)
    ha = h_ref[...] + proj
    ha_ref[...] = ha
    x2 = _ln(ha, lnw_ref[...], lnb_ref[...])
    x2_ref[...] = x2.astype(jnp.bfloat16)
    lg_ref[...] = jax.lax.dot_general(x2.astype(jnp.bfloat16), rw_ref[...],
                                      (((1,), (1,)), ((), ())),
                                      preferred_element_type=jnp.float32)


def _ffn_body(meta_ref, xs_ref, w1_ref, w2_ref, g_ref, ys_ref):
    i = pl.program_id(1)

    @pl.when(i < meta_ref[NBLK])
    def _():
        xs = xs_ref[...].astype(jnp.float32)
        h = jax.lax.dot_general(xs, w1_ref[0], (((1,), (1,)), ((), ())),
                                preferred_element_type=jnp.float32)
        h = jax.nn.gelu(h)
        o = jax.lax.dot_general(h, w2_ref[0], (((1,), (1,)), ((), ())),
                                preferred_element_type=jnp.float32)
        g = g_ref[0, 0, :]
        ys_ref[0] = o * g[:, None]


def _combine_body(ha_ref, a_ref, b_ref, c_ref, d_ref, out_ref):
    out_ref[...] = (ha_ref[...] + a_ref[...] + b_ref[...]
                    + c_ref[...] + d_ref[...])


def kernel(hidden_states, ln1_weight, ln1_bias, ln2_weight, ln2_bias,
           qkv_weight, proj_weight, router_weight, moe_w1, moe_w2):
    x = hidden_states.reshape(S, HIDDEN)
    f32 = jnp.float32

    ln1w = ln1_weight.reshape(1, HIDDEN)
    ln1b = ln1_bias.reshape(1, HIDDEN)
    ln2w = ln2_weight.reshape(1, HIDDEN)
    ln2b = ln2_bias.reshape(1, HIDDEN)

    hd = jax.ShapeDtypeStruct((NUM_HEADS, S, DH), jnp.bfloat16)
    q, k, v = pl.pallas_call(
        _ln1_qkv_body,
        grid=(S // BR,),
        in_specs=[
            pl.BlockSpec((BR, HIDDEN), lambda i: (i, 0)),
            pl.BlockSpec((3 * HIDDEN, HIDDEN), lambda i: (0, 0)),
            pl.BlockSpec((1, HIDDEN), lambda i: (0, 0)),
            pl.BlockSpec((1, HIDDEN), lambda i: (0, 0)),
        ],
        out_specs=[pl.BlockSpec((NUM_HEADS, BR, DH), lambda i: (0, i, 0))] * 3,
        out_shape=[hd, hd, hd],
    )(x, qkv_weight.astype(jnp.bfloat16), ln1w, ln1b)

    ao = pl.pallas_call(
        _attn_body,
        grid=(NUM_HEADS, S // BQ),
        in_specs=[
            pl.BlockSpec((1, BQ, DH), lambda h, i: (h, i, 0)),
            pl.BlockSpec((1, S, DH), lambda h, i: (h, 0, 0)),
            pl.BlockSpec((1, S, DH), lambda h, i: (h, 0, 0)),
        ],
        out_specs=pl.BlockSpec((1, BQ, DH), lambda h, i: (h, i, 0)),
        out_shape=jax.ShapeDtypeStruct((NUM_HEADS, S, DH), jnp.bfloat16),
    )(q, k, v)

    ha, x2b, logits = pl.pallas_call(
        _proj_ln2_body,
        grid=(S // BR,),
        in_specs=[
            pl.BlockSpec((NUM_HEADS, BR, DH), lambda i: (0, i, 0)),
            pl.BlockSpec((BR, HIDDEN), lambda i: (i, 0)),
            pl.BlockSpec((HIDDEN, HIDDEN), lambda i: (0, 0)),
            pl.BlockSpec((1, HIDDEN), lambda i: (0, 0)),
            pl.BlockSpec((1, HIDDEN), lambda i: (0, 0)),
            pl.BlockSpec((NUM_EXPERTS, HIDDEN), lambda i: (0, 0)),
        ],
        out_specs=[
            pl.BlockSpec((BR, HIDDEN), lambda i: (i, 0)),
            pl.BlockSpec((BR, HIDDEN), lambda i: (i, 0)),
            pl.BlockSpec((BR, NUM_EXPERTS), lambda i: (i, 0)),
        ],
        out_shape=[
            jax.ShapeDtypeStruct((S, HIDDEN), f32),
            jax.ShapeDtypeStruct((S, HIDDEN), jnp.bfloat16),
            jax.ShapeDtypeStruct((S, NUM_EXPERTS), f32),
        ],
    )(ao, x, proj_weight.astype(jnp.bfloat16), ln2w, ln2b,
      router_weight.astype(jnp.bfloat16))

    probs = jax.nn.softmax(logits, axis=-1)
    top_p, top_i = jax.lax.top_k(probs, TOP_K)
    flat_e = top_i.reshape(-1).astype(jnp.int32)
    onehot = (flat_e[:, None] == jnp.arange(NUM_EXPERTS)[None, :]
              ).astype(jnp.int32)
    counts = onehot.sum(0)
    nblk_e = (counts + BLK - 1) // BLK
    padded = nblk_e * BLK
    ends = jnp.cumsum(padded)
    offs = ends - padded
    rank = jnp.cumsum(onehot, axis=0) - onehot
    myrank = (rank * onehot).sum(1)
    pos = offs[flat_e] + myrank
    slot_tok = jnp.arange(NSLOT, dtype=jnp.int32) // TOP_K
    tids = jnp.zeros((NROWS,), jnp.int32).at[pos].set(slot_tok)
    gates = jnp.zeros((NROWS,), f32).at[pos].set(top_p.reshape(-1))
    blk_start = jnp.arange(NBLK, dtype=jnp.int32) * BLK
    block_expert = (blk_start[:, None] >= ends[None, :]).sum(1)
    block_expert = jnp.minimum(block_expert, NUM_EXPERTS - 1).astype(jnp.int32)
    num_used = nblk_e.sum().astype(jnp.int32)
    meta = jnp.concatenate([block_expert, num_used[None]])

    xs = x2b[tids]
    gates3 = gates.reshape(NBLK, 1, BLK)
    FC = FFN // 2
    NFC = FFN // FC

    ysp = pl.pallas_call(
        _ffn_body,
        grid_spec=pltpu.PrefetchScalarGridSpec(
            num_scalar_prefetch=1,
            grid=(NFC, NBLK),
            in_specs=[
                pl.BlockSpec((BLK, HIDDEN), lambda fc, i, m: (i, 0)),
                pl.BlockSpec((1, FC, HIDDEN), lambda fc, i, m: (m[i], fc, 0)),
                pl.BlockSpec((1, HIDDEN, FC), lambda fc, i, m: (m[i], 0, fc)),
                pl.BlockSpec((1, 1, BLK), lambda fc, i, m: (i, 0, 0)),
            ],
            out_specs=pl.BlockSpec((1, BLK, HIDDEN), lambda fc, i, m: (fc, i, 0)),
        ),
        out_shape=jax.ShapeDtypeStruct((NFC, NROWS, HIDDEN), f32),
    )(meta, xs, moe_w1, moe_w2, gates3)

    flat = ysp.reshape(NFC * NROWS, HIDDEN)
    pos2 = pos.reshape(S, TOP_K)
    allpos = jnp.concatenate(
        [pos2[:, 0], pos2[:, 1], pos2[:, 0] + NROWS, pos2[:, 1] + NROWS])
    zz = flat[allpos]
    out = pl.pallas_call(
        _combine_body,
        grid=(S // BR,),
        in_specs=[
            pl.BlockSpec((BR, HIDDEN), lambda i: (i, 0)),
            pl.BlockSpec((BR, HIDDEN), lambda i: (i, 0)),
            pl.BlockSpec((BR, HIDDEN), lambda i: (S // BR + i, 0)),
            pl.BlockSpec((BR, HIDDEN), lambda i: (2 * (S // BR) + i, 0)),
            pl.BlockSpec((BR, HIDDEN), lambda i: (3 * (S // BR) + i, 0)),
        ],
        out_specs=pl.BlockSpec((BR, HIDDEN), lambda i: (i, 0)),
        out_shape=jax.ShapeDtypeStruct((S, HIDDEN), f32),
    )(ha, zz, zz, zz, zz)

    return out.reshape(S, 1, HIDDEN)


# speedup vs baseline: 1.1265x; 1.0250x over previous
import functools

import jax
import jax.numpy as jnp
from jax.experimental import pallas as pl
from jax.experimental.pallas import tpu as pltpu

NUM_HEADS = 16
NUM_EXPERTS = 8
TOP_K = 2
HIDDEN = 1024
FFN = 4096
EPS = 1e-5

S = 2048
DH = HIDDEN // NUM_HEADS
BQ = 512
BK = 512
BR = 256
BLK = 256
NSLOT = S * TOP_K
NROWS = ((NSLOT + NUM_EXPERTS * (BLK - 1) + BLK - 1) // BLK) * BLK
NBLK = NROWS // BLK


def _ln(x, w, b):
    mu = jnp.mean(x, axis=-1, keepdims=True)
    var = jnp.mean((x - mu) ** 2, axis=-1, keepdims=True)
    return (x - mu) / jnp.sqrt(var + EPS) * w + b


def _ln1_qkv_body(x_ref, w_ref, lnw_ref, lnb_ref, q_ref, k_ref, v_ref):
    x = x_ref[...]
    ln = _ln(x, lnw_ref[...], lnb_ref[...]).astype(jnp.bfloat16)
    qkv = jax.lax.dot_general(
        ln, w_ref[...], (((1,), (1,)), ((), ())),
        preferred_element_type=jnp.float32).astype(jnp.bfloat16)
    for ref, off in ((q_ref, 0), (k_ref, HIDDEN), (v_ref, 2 * HIDDEN)):
        part = qkv[:, off:off + HIDDEN]
        ref[...] = part.reshape(BR, NUM_HEADS, DH).transpose(1, 0, 2)


def _attn_body(q_ref, k_ref, v_ref, o_ref):
    qi = pl.program_id(1)
    q = q_ref[0] * jnp.bfloat16(1.0 / DH ** 0.5)

    def tile(t, carry):
        acc, l = carry
        kt = k_ref[0, pl.ds(t * BK, BK), :]
        s = jax.lax.dot_general(q, kt, (((1,), (1,)), ((), ())),
                                preferred_element_type=jnp.float32)
        row = qi * BQ + jax.lax.broadcasted_iota(jnp.int32, (BQ, BK), 0)
        col = t * BK + jax.lax.broadcasted_iota(jnp.int32, (BQ, BK), 1)
        p = jnp.where(col <= row, jnp.exp(s), 0.0)
        l = l + jnp.sum(p, axis=1, keepdims=True)
        vt = v_ref[0, pl.ds(t * BK, BK), :]
        acc = acc + jax.lax.dot_general(p.astype(jnp.bfloat16), vt,
                                        (((1,), (0,)), ((), ())),
                                        preferred_element_type=jnp.float32)
        return acc, l

    acc0 = jnp.zeros((BQ, DH), jnp.float32)
    l0 = jnp.zeros((BQ, 1), jnp.float32)
    acc, l = jax.lax.fori_loop(0, qi + 1, tile, (acc0, l0))
    o_ref[0] = (acc / l).astype(jnp.bfloat16)


def _proj_ln2_body(ao_ref, h_ref, pw_ref, lnw_ref, lnb_ref, rw_ref,
                   ha_ref, x2_ref, lg_ref):
    ao = ao_ref[...].transpose(1, 0, 2).reshape(BR, HIDDEN)
    proj = jax.lax.dot_general(ao, pw_ref[...],
                               (((1,), (1,)), ((), ())),
                               preferred_element_type=jnp.float32)
    ha = h_ref[...] + proj
    ha_ref[...] = ha
    x2 = _ln(ha, lnw_ref[...], lnb_ref[...])
    x2_ref[...] = x2.astype(jnp.bfloat16)
    lg_ref[...] = jax.lax.dot_general(x2.astype(jnp.bfloat16), rw_ref[...],
                                      (((1,), (1,)), ((), ())),
                                      preferred_element_type=jnp.float32)


def _ffn_body(meta_ref, xs_ref, w1_ref, w2_ref, g_ref, ys_ref):
    i = pl.program_id(1)

    @pl.when(i < meta_ref[NBLK])
    def _():
        xs = xs_ref[...].astype(jnp.float32)
        h = jax.lax.dot_general(xs, w1_ref[0], (((1,), (1,)), ((), ())),
                                preferred_element_type=jnp.float32)
        h = jax.nn.gelu(h)
        o = jax.lax.dot_general(h, w2_ref[0], (((1,), (1,)), ((), ())),
                                preferred_element_type=jnp.float32)
        g = g_ref[0, 0, :]
        ys_ref[0] = o * g[:, None]


def _combine_body(ha_ref, a_ref, b_ref, c_ref, d_ref, out_ref):
    out_ref[...] = (ha_ref[...] + a_ref[...] + b_ref[...]
                    + c_ref[...] + d_ref[...])


def kernel(hidden_states, ln1_weight, ln1_bias, ln2_weight, ln2_bias,
           qkv_weight, proj_weight, router_weight, moe_w1, moe_w2):
    x = hidden_states.reshape(S, HIDDEN)
    f32 = jnp.float32

    ln1w = ln1_weight.reshape(1, HIDDEN)
    ln1b = ln1_bias.reshape(1, HIDDEN)
    ln2w = ln2_weight.reshape(1, HIDDEN)
    ln2b = ln2_bias.reshape(1, HIDDEN)

    hd = jax.ShapeDtypeStruct((NUM_HEADS, S, DH), jnp.bfloat16)
    q, k, v = pl.pallas_call(
        _ln1_qkv_body,
        grid=(S // BR,),
        in_specs=[
            pl.BlockSpec((BR, HIDDEN), lambda i: (i, 0)),
            pl.BlockSpec((3 * HIDDEN, HIDDEN), lambda i: (0, 0)),
            pl.BlockSpec((1, HIDDEN), lambda i: (0, 0)),
            pl.BlockSpec((1, HIDDEN), lambda i: (0, 0)),
        ],
        out_specs=[pl.BlockSpec((NUM_HEADS, BR, DH), lambda i: (0, i, 0))] * 3,
        out_shape=[hd, hd, hd],
    )(x, qkv_weight.astype(jnp.bfloat16), ln1w, ln1b)

    ao = pl.pallas_call(
        _attn_body,
        grid=(NUM_HEADS, S // BQ),
        in_specs=[
            pl.BlockSpec((1, BQ, DH), lambda h, i: (h, i, 0)),
            pl.BlockSpec((1, S, DH), lambda h, i: (h, 0, 0)),
            pl.BlockSpec((1, S, DH), lambda h, i: (h, 0, 0)),
        ],
        out_specs=pl.BlockSpec((1, BQ, DH), lambda h, i: (h, i, 0)),
        out_shape=jax.ShapeDtypeStruct((NUM_HEADS, S, DH), jnp.bfloat16),
    )(q, k, v)

    ha, x2b, logits = pl.pallas_call(
        _proj_ln2_body,
        grid=(S // BR,),
        in_specs=[
            pl.BlockSpec((NUM_HEADS, BR, DH), lambda i: (0, i, 0)),
            pl.BlockSpec((BR, HIDDEN), lambda i: (i, 0)),
            pl.BlockSpec((HIDDEN, HIDDEN), lambda i: (0, 0)),
            pl.BlockSpec((1, HIDDEN), lambda i: (0, 0)),
            pl.BlockSpec((1, HIDDEN), lambda i: (0, 0)),
            pl.BlockSpec((NUM_EXPERTS, HIDDEN), lambda i: (0, 0)),
        ],
        out_specs=[
            pl.BlockSpec((BR, HIDDEN), lambda i: (i, 0)),
            pl.BlockSpec((BR, HIDDEN), lambda i: (i, 0)),
            pl.BlockSpec((BR, NUM_EXPERTS), lambda i: (i, 0)),
        ],
        out_shape=[
            jax.ShapeDtypeStruct((S, HIDDEN), f32),
            jax.ShapeDtypeStruct((S, HIDDEN), jnp.bfloat16),
            jax.ShapeDtypeStruct((S, NUM_EXPERTS), f32),
        ],
    )(ao, x, proj_weight.astype(jnp.bfloat16), ln2w, ln2b,
      router_weight.astype(jnp.bfloat16))

    probs = jax.nn.softmax(logits, axis=-1)
    top_p, top_i = jax.lax.top_k(probs, TOP_K)
    flat_e = top_i.reshape(-1).astype(jnp.int32)
    onehot = (flat_e[:, None] == jnp.arange(NUM_EXPERTS)[None, :]
              ).astype(jnp.int32)
    counts = onehot.sum(0)
    nblk_e = (counts + BLK - 1) // BLK
    padded = nblk_e * BLK
    ends = jnp.cumsum(padded)
    offs = ends - padded
    rank = jnp.cumsum(onehot, axis=0) - onehot
    myrank = (rank * onehot).sum(1)
    pos = offs[flat_e] + myrank
    slot_tok = jnp.arange(NSLOT, dtype=jnp.int32) // TOP_K
    tids = jnp.zeros((NROWS,), jnp.int32).at[pos].set(slot_tok)
    gates = jnp.zeros((NROWS,), f32).at[pos].set(top_p.reshape(-1))
    blk_start = jnp.arange(NBLK, dtype=jnp.int32) * BLK
    block_expert = (blk_start[:, None] >= ends[None, :]).sum(1)
    block_expert = jnp.minimum(block_expert, NUM_EXPERTS - 1).astype(jnp.int32)
    num_used = nblk_e.sum().astype(jnp.int32)
    meta = jnp.concatenate([block_expert, num_used[None]])

    xs = x2b[tids]
    gates3 = gates.reshape(NBLK, 1, BLK)
    FC = FFN // 2
    NFC = FFN // FC

    ysp = pl.pallas_call(
        _ffn_body,
        grid_spec=pltpu.PrefetchScalarGridSpec(
            num_scalar_prefetch=1,
            grid=(NFC, NBLK),
            in_specs=[
                pl.BlockSpec((BLK, HIDDEN), lambda fc, i, m: (i, 0)),
                pl.BlockSpec((1, FC, HIDDEN), lambda fc, i, m: (m[i], fc, 0)),
                pl.BlockSpec((1, HIDDEN, FC), lambda fc, i, m: (m[i], 0, fc)),
                pl.BlockSpec((1, 1, BLK), lambda fc, i, m: (i, 0, 0)),
            ],
            out_specs=pl.BlockSpec((1, BLK, HIDDEN), lambda fc, i, m: (fc, i, 0)),
        ),
        out_shape=jax.ShapeDtypeStruct((NFC, NROWS, HIDDEN), f32),
    )(meta, xs, moe_w1, moe_w2, gates3)

    flat = ysp.reshape(NFC * NROWS, HIDDEN)
    pos2 = pos.reshape(S, TOP_K)
    allpos = jnp.concatenate(
        [pos2[:, 0], pos2[:, 1], pos2[:, 0] + NROWS, pos2[:, 1] + NROWS])
    zz = flat[allpos]
    out = pl.pallas_call(
        _combine_body,
        grid=(S // BR,),
        in_specs=[
            pl.BlockSpec((BR, HIDDEN), lambda i: (i, 0)),
            pl.BlockSpec((BR, HIDDEN), lambda i: (i, 0)),
            pl.BlockSpec((BR, HIDDEN), lambda i: (S // BR + i, 0)),
            pl.BlockSpec((BR, HIDDEN), lambda i: (2 * (S // BR) + i, 0)),
            pl.BlockSpec((BR, HIDDEN), lambda i: (3 * (S // BR) + i, 0)),
        ],
        out_specs=pl.BlockSpec((BR, HIDDEN), lambda i: (i, 0)),
        out_shape=jax.ShapeDtypeStruct((S, HIDDEN), f32),
    )(ha, zz, zz, zz, zz)

    return out.reshape(S, 1, HIDDEN)


# speedup vs baseline: 1.1408x; 1.0127x over previous
import functools

import jax
import jax.numpy as jnp
from jax import lax
from jax.experimental import pallas as pl
from jax.experimental.pallas import tpu as pltpu
from jax.experimental.pallas import tpu_sc as plsc

NUM_HEADS = 16
NUM_EXPERTS = 8
TOP_K = 2
HIDDEN = 1024
FFN = 4096
EPS = 1e-5

S = 2048
DH = HIDDEN // NUM_HEADS
BQ = 512
BK = 512
BR = 256
BLK = 256
NSLOT = S * TOP_K
NROWS = ((NSLOT + NUM_EXPERTS * (BLK - 1) + BLK - 1) // BLK) * BLK
NBLK = NROWS // BLK


def _ln(x, w, b):
    mu = jnp.mean(x, axis=-1, keepdims=True)
    var = jnp.mean((x - mu) ** 2, axis=-1, keepdims=True)
    return (x - mu) / jnp.sqrt(var + EPS) * w + b


def _ln1_qkv_body(x_ref, w_ref, lnw_ref, lnb_ref, q_ref, k_ref, v_ref):
    x = x_ref[...]
    ln = _ln(x, lnw_ref[...], lnb_ref[...]).astype(jnp.bfloat16)
    qkv = jax.lax.dot_general(
        ln, w_ref[...], (((1,), (1,)), ((), ())),
        preferred_element_type=jnp.float32).astype(jnp.bfloat16)
    for ref, off in ((q_ref, 0), (k_ref, HIDDEN), (v_ref, 2 * HIDDEN)):
        part = qkv[:, off:off + HIDDEN]
        ref[...] = part.reshape(BR, NUM_HEADS, DH).transpose(1, 0, 2)


def _attn_body(q_ref, k_ref, v_ref, o_ref):
    qi = pl.program_id(1)
    q = q_ref[0] * jnp.bfloat16(1.0 / DH ** 0.5)

    def tile(t, carry):
        acc, l = carry
        kt = k_ref[0, pl.ds(t * BK, BK), :]
        s = jax.lax.dot_general(q, kt, (((1,), (1,)), ((), ())),
                                preferred_element_type=jnp.float32)
        row = qi * BQ + jax.lax.broadcasted_iota(jnp.int32, (BQ, BK), 0)
        col = t * BK + jax.lax.broadcasted_iota(jnp.int32, (BQ, BK), 1)
        p = jnp.where(col <= row, jnp.exp(s), 0.0)
        l = l + jnp.sum(p, axis=1, keepdims=True)
        vt = v_ref[0, pl.ds(t * BK, BK), :]
        acc = acc + jax.lax.dot_general(p.astype(jnp.bfloat16), vt,
                                        (((1,), (0,)), ((), ())),
                                        preferred_element_type=jnp.float32)
        return acc, l

    acc0 = jnp.zeros((BQ, DH), jnp.float32)
    l0 = jnp.zeros((BQ, 1), jnp.float32)
    acc, l = jax.lax.fori_loop(0, qi + 1, tile, (acc0, l0))
    o_ref[0] = (acc / l).astype(jnp.bfloat16)


def _proj_ln2_body(ao_ref, h_ref, pw_ref, lnw_ref, lnb_ref, rw_ref,
                   ha_ref, x2_ref, lg_ref):
    ao = ao_ref[...].transpose(1, 0, 2).reshape(BR, HIDDEN)
    proj = jax.lax.dot_general(ao, pw_ref[...],
                               (((1,), (1,)), ((), ())),
                               preferred_element_type=jnp.float32)
    ha = h_ref[...] + proj
    ha_ref[...] = ha
    x2 = _ln(ha, lnw_ref[...], lnb_ref[...])
    x2_ref[...] = x2.astype(jnp.bfloat16)
    lg_ref[...] = jax.lax.dot_general(x2.astype(jnp.bfloat16), rw_ref[...],
                                      (((1,), (1,)), ((), ())),
                                      preferred_element_type=jnp.float32)


def _ffn_body(meta_ref, xs_ref, w1_ref, w2_ref, g_ref, ys_ref):
    i = pl.program_id(1)

    @pl.when(i < meta_ref[NBLK])
    def _():
        xs = xs_ref[...].astype(jnp.float32)
        h = jax.lax.dot_general(xs, w1_ref[0], (((1,), (1,)), ((), ())),
                                preferred_element_type=jnp.float32)
        h = jax.nn.gelu(h)
        o = jax.lax.dot_general(h, w2_ref[0], (((1,), (1,)), ((), ())),
                                preferred_element_type=jnp.float32)
        g = g_ref[0, 0, :]
        ys_ref[0] = o * g[:, None]


def _sc_row_gather(table, idx):
    info = plsc.get_sparse_core_info()
    nw = info.num_cores * info.num_subcores
    b = idx.shape[0]
    b_per_w = b // nw
    ch = 64
    nch = b_per_w // ch
    mesh = plsc.VectorSubcoreMesh(core_axis_name="c", subcore_axis_name="s")

    @functools.partial(
        pl.kernel, mesh=mesh,
        out_type=jax.ShapeDtypeStruct((b, table.shape[1]), table.dtype),
        scratch_types=[
            pltpu.VMEM((b_per_w,), jnp.int32),
            pltpu.VMEM((ch, table.shape[1]), table.dtype),
            pltpu.SemaphoreType.DMA,
        ],
    )
    def k(table_hbm, idx_hbm, out_hbm, idx_v, rows_v, sem):
        wid = lax.axis_index("s") * info.num_cores + lax.axis_index("c")
        base = wid * b_per_w
        pltpu.sync_copy(idx_hbm.at[pl.ds(base, b_per_w)], idx_v)
        for ci in range(nch):
            pltpu.async_copy(
                table_hbm.at[idx_v.at[pl.ds(ci * ch, ch)]], rows_v, sem).wait()
            pltpu.sync_copy(rows_v, out_hbm.at[pl.ds(base + ci * ch, ch)])

    return k(table, idx)


def _combine_body(ha_ref, a_ref, b_ref, c_ref, d_ref, out_ref):
    out_ref[...] = (ha_ref[...] + a_ref[...] + b_ref[...]
                    + c_ref[...] + d_ref[...])


def kernel(hidden_states, ln1_weight, ln1_bias, ln2_weight, ln2_bias,
           qkv_weight, proj_weight, router_weight, moe_w1, moe_w2):
    x = hidden_states.reshape(S, HIDDEN)
    f32 = jnp.float32

    ln1w = ln1_weight.reshape(1, HIDDEN)
    ln1b = ln1_bias.reshape(1, HIDDEN)
    ln2w = ln2_weight.reshape(1, HIDDEN)
    ln2b = ln2_bias.reshape(1, HIDDEN)

    hd = jax.ShapeDtypeStruct((NUM_HEADS, S, DH), jnp.bfloat16)
    q, k, v = pl.pallas_call(
        _ln1_qkv_body,
        grid=(S // BR,),
        in_specs=[
            pl.BlockSpec((BR, HIDDEN), lambda i: (i, 0)),
            pl.BlockSpec((3 * HIDDEN, HIDDEN), lambda i: (0, 0)),
            pl.BlockSpec((1, HIDDEN), lambda i: (0, 0)),
            pl.BlockSpec((1, HIDDEN), lambda i: (0, 0)),
        ],
        out_specs=[pl.BlockSpec((NUM_HEADS, BR, DH), lambda i: (0, i, 0))] * 3,
        out_shape=[hd, hd, hd],
    )(x, qkv_weight.astype(jnp.bfloat16), ln1w, ln1b)

    ao = pl.pallas_call(
        _attn_body,
        grid=(NUM_HEADS, S // BQ),
        in_specs=[
            pl.BlockSpec((1, BQ, DH), lambda h, i: (h, i, 0)),
            pl.BlockSpec((1, S, DH), lambda h, i: (h, 0, 0)),
            pl.BlockSpec((1, S, DH), lambda h, i: (h, 0, 0)),
        ],
        out_specs=pl.BlockSpec((1, BQ, DH), lambda h, i: (h, i, 0)),
        out_shape=jax.ShapeDtypeStruct((NUM_HEADS, S, DH), jnp.bfloat16),
    )(q, k, v)

    ha, x2b, logits = pl.pallas_call(
        _proj_ln2_body,
        grid=(S // BR,),
        in_specs=[
            pl.BlockSpec((NUM_HEADS, BR, DH), lambda i: (0, i, 0)),
            pl.BlockSpec((BR, HIDDEN), lambda i: (i, 0)),
            pl.BlockSpec((HIDDEN, HIDDEN), lambda i: (0, 0)),
            pl.BlockSpec((1, HIDDEN), lambda i: (0, 0)),
            pl.BlockSpec((1, HIDDEN), lambda i: (0, 0)),
            pl.BlockSpec((NUM_EXPERTS, HIDDEN), lambda i: (0, 0)),
        ],
        out_specs=[
            pl.BlockSpec((BR, HIDDEN), lambda i: (i, 0)),
            pl.BlockSpec((BR, HIDDEN), lambda i: (i, 0)),
            pl.BlockSpec((BR, NUM_EXPERTS), lambda i: (i, 0)),
        ],
        out_shape=[
            jax.ShapeDtypeStruct((S, HIDDEN), f32),
            jax.ShapeDtypeStruct((S, HIDDEN), jnp.bfloat16),
            jax.ShapeDtypeStruct((S, NUM_EXPERTS), f32),
        ],
    )(ao, x, proj_weight.astype(jnp.bfloat16), ln2w, ln2b,
      router_weight.astype(jnp.bfloat16))

    probs = jax.nn.softmax(logits, axis=-1)
    top_p, top_i = jax.lax.top_k(probs, TOP_K)
    flat_e = top_i.reshape(-1).astype(jnp.int32)
    onehot = (flat_e[:, None] == jnp.arange(NUM_EXPERTS)[None, :]
              ).astype(jnp.int32)
    counts = onehot.sum(0)
    nblk_e = (counts + BLK - 1) // BLK
    padded = nblk_e * BLK
    ends = jnp.cumsum(padded)
    offs = ends - padded
    rank = jnp.cumsum(onehot, axis=0) - onehot
    myrank = (rank * onehot).sum(1)
    pos = offs[flat_e] + myrank
    slot_tok = jnp.arange(NSLOT, dtype=jnp.int32) // TOP_K
    tids = jnp.zeros((NROWS,), jnp.int32).at[pos].set(slot_tok)
    gates = jnp.zeros((NROWS,), f32).at[pos].set(top_p.reshape(-1))
    blk_start = jnp.arange(NBLK, dtype=jnp.int32) * BLK
    block_expert = (blk_start[:, None] >= ends[None, :]).sum(1)
    block_expert = jnp.minimum(block_expert, NUM_EXPERTS - 1).astype(jnp.int32)
    num_used = nblk_e.sum().astype(jnp.int32)
    meta = jnp.concatenate([block_expert, num_used[None]])

    xs = x2b[tids]
    gates3 = gates.reshape(NBLK, 1, BLK)
    FC = FFN // 2
    NFC = FFN // FC

    ysp = pl.pallas_call(
        _ffn_body,
        grid_spec=pltpu.PrefetchScalarGridSpec(
            num_scalar_prefetch=1,
            grid=(NFC, NBLK),
            in_specs=[
                pl.BlockSpec((BLK, HIDDEN), lambda fc, i, m: (i, 0)),
                pl.BlockSpec((1, FC, HIDDEN), lambda fc, i, m: (m[i], fc, 0)),
                pl.BlockSpec((1, HIDDEN, FC), lambda fc, i, m: (m[i], 0, fc)),
                pl.BlockSpec((1, 1, BLK), lambda fc, i, m: (i, 0, 0)),
            ],
            out_specs=pl.BlockSpec((1, BLK, HIDDEN), lambda fc, i, m: (fc, i, 0)),
        ),
        out_shape=jax.ShapeDtypeStruct((NFC, NROWS, HIDDEN), f32),
    )(meta, xs, moe_w1, moe_w2, gates3)

    flat = ysp.reshape(NFC * NROWS, HIDDEN)
    pos2 = pos.reshape(S, TOP_K)
    allpos = jnp.concatenate(
        [pos2[:, 0], pos2[:, 1], pos2[:, 0] + NROWS, pos2[:, 1] + NROWS])
    zz = _sc_row_gather(flat, allpos)
    out = pl.pallas_call(
        _combine_body,
        grid=(S // BR,),
        in_specs=[
            pl.BlockSpec((BR, HIDDEN), lambda i: (i, 0)),
            pl.BlockSpec((BR, HIDDEN), lambda i: (i, 0)),
            pl.BlockSpec((BR, HIDDEN), lambda i: (S // BR + i, 0)),
            pl.BlockSpec((BR, HIDDEN), lambda i: (2 * (S // BR) + i, 0)),
            pl.BlockSpec((BR, HIDDEN), lambda i: (3 * (S // BR) + i, 0)),
        ],
        out_specs=pl.BlockSpec((BR, HIDDEN), lambda i: (i, 0)),
        out_shape=jax.ShapeDtypeStruct((S, HIDDEN), f32),
    )(ha, zz, zz, zz, zz)

    return out.reshape(S, 1, HIDDEN)
